# baseline (speedup 1.0000x reference)
import functools

import jax
import jax.numpy as jnp
from jax import lax
from jax.experimental import pallas as pl
from jax.experimental.pallas import tpu as pltpu
from jax.experimental.pallas import tpu_sc as plsc

B, S = 4096, 200
VOCAB = 30000
D = 768
K1 = 1.2
BB = 0.75
EPS = 1e-10

NC, NS, L = 2, 16, 16
NW = NC * NS
DPW = B // NW
GROUP = 16
CHUNKS = ((0, 56), (56, 56), (112, 56), (168, 32))
CMAX = 56


RBLK = 256
WBLK = 32


def _weights_body(ids_ref, mask_ref, w_ref):
    ids = ids_ref[...]
    msk = mask_ref[...]
    valid = (msk == 1) & (ids > 100) & (ids < VOCAB)
    vf = valid.astype(jnp.float32)
    doc_len = jnp.sum(vf, axis=1, keepdims=True)
    denom = jnp.maximum(1.0 + BB * (doc_len / 100.0 - 1.0), 0.5)
    safe = jnp.where(valid, ids, -1)
    eq = safe[:, :, None] == safe[:, None, :]
    c = jnp.sum(eq.astype(jnp.float32), axis=2)
    w_ref[...] = jnp.where(valid, (K1 + 1.0) / (c + K1 * denom), 0.0)


def _weights(input_ids, attention_mask):
    return pl.pallas_call(
        _weights_body,
        grid=(B // WBLK,),
        in_specs=[
            pl.BlockSpec((WBLK, S), lambda i: (i, 0)),
            pl.BlockSpec((WBLK, S), lambda i: (i, 0)),
        ],
        out_specs=pl.BlockSpec((WBLK, S), lambda i: (i, 0)),
        out_shape=jax.ShapeDtypeStruct((B, S), jnp.float32),
    )(input_ids, attention_mask)


PBLK = 512


def _pack_body(wlo_ref, whi_ref, wt_ref, ulo_ref, uhi_ref):
    @pl.when(pl.program_id(0) == 0)
    def _():
        ulo_ref[...] = jnp.zeros_like(ulo_ref)
        uhi_ref[...] = jnp.zeros_like(uhi_ref)

    wlo = wlo_ref[...]
    whi = whi_ref[...]
    col = pl.program_id(0) * PBLK + jax.lax.broadcasted_iota(
        jnp.int32, (D // 2, PBLK), 1)
    mask = col < VOCAB
    ulo_ref[...] += jnp.sum(jnp.where(mask, wlo, 0.0), axis=1, keepdims=True)
    uhi_ref[...] += jnp.sum(jnp.where(mask, whi, 0.0), axis=1, keepdims=True)
    lo = lax.bitcast_convert_type(
        wlo.astype(jnp.bfloat16), jnp.uint16).astype(jnp.int32)
    hi = lax.bitcast_convert_type(
        whi.astype(jnp.bfloat16), jnp.uint16).astype(jnp.int32)
    word = lo | (hi << 16)
    wt_ref[...] = word.T


def _packtable(W):
    return pl.pallas_call(
        _pack_body,
        grid=(pl.cdiv(VOCAB, PBLK),),
        in_specs=[
            pl.BlockSpec((D // 2, PBLK), lambda i: (0, i)),
            pl.BlockSpec((D // 2, PBLK), lambda i: (1, i)),
        ],
        out_specs=[
            pl.BlockSpec((PBLK, D // 2), lambda i: (i, 0)),
            pl.BlockSpec((D // 2, 1), lambda i: (0, 0)),
            pl.BlockSpec((D // 2, 1), lambda i: (0, 0)),
        ],
        out_shape=[
            jax.ShapeDtypeStruct((VOCAB, D // 2), jnp.int32),
            jax.ShapeDtypeStruct((D // 2, 1), jnp.float32),
            jax.ShapeDtypeStruct((D // 2, 1), jnp.float32),
        ],
    )(W, W)


def _sc_body(wt_hbm, ids_hbm, w_hbm, out_hbm,
             ids_v, w_v, rows0, rows1, acc, sem0, sem1):
    wid = lax.axis_index("s") * NC + lax.axis_index("c")
    base = wid * DPW
    bufs = (rows0, rows1)
    sems = (sem0, sem1)

    def group_body(g, _):
        g0 = base + g * GROUP
        tok0 = pl.multiple_of(g0 * S, 8)
        pltpu.sync_copy(ids_hbm.at[pl.ds(tok0, GROUP * S)], ids_v)
        pltpu.sync_copy(w_hbm.at[pl.ds(tok0, GROUP * S)], w_v)

        def doc_body(d, _):
            for i in range(D // L):
                acc[pl.ds(i * L, L)] = jnp.zeros((L,), jnp.float32)

            def start(ci):
                off, sz = CHUNKS[ci]
                return pltpu.async_copy(
                    wt_hbm.at[ids_v.at[pl.ds(pl.multiple_of(d * S + off, 8), sz)]],
                    bufs[ci % 2].at[pl.ds(0, sz)],
                    sems[ci % 2],
                )

            descs = [None] * len(CHUNKS)
            descs[0] = start(0)
            for ci, (off, sz) in enumerate(CHUNKS):
                if ci + 1 < len(CHUNKS):
                    descs[ci + 1] = start(ci + 1)
                descs[ci].wait()
                buf = bufs[ci % 2]

                ngrp = D // (2 * L)
                halfg = ngrp // 2
                for h in range(2):
                    def row_body(r2, accs):
                        idx = jnp.full(
                            (L,), d * S + off, jnp.int32) + 2 * r2
                        wb0 = plsc.load_gather(w_v, [idx])
                        wb1 = plsc.load_gather(w_v, [idx + 1])
                        wbb0 = plsc.pack(
                            wb0, wb0, format=plsc.PackFormat.INTERLEAVED)
                        wbb1 = plsc.pack(
                            wb1, wb1, format=plsc.PackFormat.INTERLEAVED)
                        out = []
                        for i in range(halfg):
                            g = h * halfg + i
                            x0 = plsc.bitcast(
                                buf[2 * r2, pl.ds(g * L, L)], jnp.bfloat16)
                            x1 = plsc.bitcast(
                                buf[2 * r2 + 1, pl.ds(g * L, L)],
                                jnp.bfloat16)
                            p = x0 * wbb0 + x1 * wbb1
                            lo, hi = plsc.unpack(
                                p, format=plsc.PackFormat.INTERLEAVED)
                            out.append(accs[2 * i] + lo)
                            out.append(accs[2 * i + 1] + hi)
                        return tuple(out)

                    zero = jnp.zeros((L,), jnp.float32)
                    accs = lax.fori_loop(0, sz // 2, row_body, (zero,) * ngrp)
                    for i in range(halfg):
                        g = h * halfg + i
                        plsc.addupdate(
                            acc.at[pl.ds(g * L, L)], accs[2 * i])
                        plsc.addupdate(
                            acc.at[pl.ds(D // 2 + g * L, L)], accs[2 * i + 1])

            pltpu.sync_copy(
                acc, out_hbm.at[pl.ds(pl.multiple_of((g0 + d) * D, 8), D)])
            return 0

        lax.fori_loop(0, GROUP, doc_body, 0)
        return 0

    lax.fori_loop(0, DPW // GROUP, group_body, 0)


def _sc_accumulate(wt, input_ids, w):
    mesh = plsc.VectorSubcoreMesh(
        core_axis_name="c", subcore_axis_name="s",
        num_cores=NC, num_subcores=NS,
    )
    f = functools.partial(
        pl.kernel,
        out_type=jax.ShapeDtypeStruct((B * D,), jnp.float32),
        mesh=mesh,
        compiler_params=pltpu.CompilerParams(needs_layout_passes=False),
        scratch_types=[
            pltpu.VMEM((GROUP * S,), jnp.int32),
            pltpu.VMEM((GROUP * S,), jnp.float32),
            pltpu.VMEM((CMAX, D // 2), jnp.int32),
            pltpu.VMEM((CMAX, D // 2), jnp.int32),
            pltpu.VMEM((D,), jnp.float32),
            pltpu.SemaphoreType.DMA,
            pltpu.SemaphoreType.DMA,
        ],
    )(_sc_body)
    return f(wt, input_ids.reshape(-1), w.reshape(-1)).reshape(B, D)


def _norm_body(a_ref, u_ref, o_ref):
    t = a_ref[...] + EPS * u_ref[...]
    ss = jnp.sum(t * t, axis=1, keepdims=True)
    n = jnp.maximum(jnp.sqrt(ss), 1e-12)
    o_ref[...] = t / n


def _normalize(acc, u):
    return pl.pallas_call(
        _norm_body,
        grid=(B // RBLK,),
        in_specs=[
            pl.BlockSpec((RBLK, D), lambda i: (i, 0)),
            pl.BlockSpec((1, D), lambda i: (0, 0)),
        ],
        out_specs=pl.BlockSpec((RBLK, D), lambda i: (i, 0)),
        out_shape=jax.ShapeDtypeStruct((B, D), jnp.float32),
    )(acc, u)


def kernel(input_ids, attention_mask, W):
    wt, ulo, uhi = _packtable(W)
    w = _weights(input_ids.astype(jnp.int32), attention_mask.astype(jnp.int32))
    u = jnp.concatenate([ulo, uhi], axis=0).reshape(1, D)
    acc = _sc_accumulate(wt, input_ids.astype(jnp.int32), w)
    return _normalize(acc, u)


# speedup vs baseline: 1.1368x; 1.1368x over previous
import functools

import jax
import jax.numpy as jnp
from jax import lax
from jax.experimental import pallas as pl
from jax.experimental.pallas import tpu as pltpu
from jax.experimental.pallas import tpu_sc as plsc

B, S = 4096, 200
VOCAB = 30000
D = 768
K1 = 1.2
BB = 0.75
EPS = 1e-10

NC, NS, L = 2, 16, 16
NW = NC * NS
DPW = B // NW
GROUP = 16
CHUNKS = ((0, 56), (56, 56), (112, 56), (168, 32))
CMAX = 56


RBLK = 256
WBLK = 32


def _weights_body(ids_ref, mask_ref, w_ref):
    ids = ids_ref[...]
    msk = mask_ref[...]
    valid = (msk == 1) & (ids > 100) & (ids < VOCAB)
    vf = valid.astype(jnp.float32)
    doc_len = jnp.sum(vf, axis=1, keepdims=True)
    denom = jnp.maximum(1.0 + BB * (doc_len / 100.0 - 1.0), 0.5)
    safe = jnp.where(valid, ids, -1)
    eq = safe[:, :, None] == safe[:, None, :]
    c = jnp.sum(eq.astype(jnp.float32), axis=2)
    w_ref[...] = jnp.where(valid, (K1 + 1.0) / (c + K1 * denom), 0.0)


def _weights(input_ids, attention_mask):
    return pl.pallas_call(
        _weights_body,
        grid=(input_ids.shape[0] // WBLK,),
        in_specs=[
            pl.BlockSpec((WBLK, S), lambda i: (i, 0)),
            pl.BlockSpec((WBLK, S), lambda i: (i, 0)),
        ],
        out_specs=pl.BlockSpec((WBLK, S), lambda i: (i, 0)),
        out_shape=jax.ShapeDtypeStruct(input_ids.shape, jnp.float32),
    )(input_ids, attention_mask)


PBLK = 512


def _pack_body(wlo_ref, whi_ref, wt_ref, ulo_ref, uhi_ref):
    @pl.when(pl.program_id(0) == 0)
    def _():
        ulo_ref[...] = jnp.zeros_like(ulo_ref)
        uhi_ref[...] = jnp.zeros_like(uhi_ref)

    wlo = wlo_ref[...]
    whi = whi_ref[...]
    col = pl.program_id(0) * PBLK + jax.lax.broadcasted_iota(
        jnp.int32, (D // 2, PBLK), 1)
    mask = col < VOCAB
    ulo_ref[...] += jnp.sum(jnp.where(mask, wlo, 0.0), axis=1, keepdims=True)
    uhi_ref[...] += jnp.sum(jnp.where(mask, whi, 0.0), axis=1, keepdims=True)
    lo = lax.bitcast_convert_type(
        wlo.astype(jnp.bfloat16), jnp.uint16).astype(jnp.int32)
    hi = lax.bitcast_convert_type(
        whi.astype(jnp.bfloat16), jnp.uint16).astype(jnp.int32)
    word = lo | (hi << 16)
    wt_ref[...] = word.T


def _packtable(W):
    return pl.pallas_call(
        _pack_body,
        grid=(pl.cdiv(VOCAB, PBLK),),
        in_specs=[
            pl.BlockSpec((D // 2, PBLK), lambda i: (0, i)),
            pl.BlockSpec((D // 2, PBLK), lambda i: (1, i)),
        ],
        out_specs=[
            pl.BlockSpec((PBLK, D // 2), lambda i: (i, 0)),
            pl.BlockSpec((D // 2, 1), lambda i: (0, 0)),
            pl.BlockSpec((D // 2, 1), lambda i: (0, 0)),
        ],
        out_shape=[
            jax.ShapeDtypeStruct((VOCAB, D // 2), jnp.int32),
            jax.ShapeDtypeStruct((D // 2, 1), jnp.float32),
            jax.ShapeDtypeStruct((D // 2, 1), jnp.float32),
        ],
    )(W, W)


def _make_sc_body(dpw):
  def _sc_body(wt_hbm, ids_hbm, w_hbm, out_hbm,
               ids_v, w_v, rows0, rows1, acc, sem0, sem1):
    wid = lax.axis_index("s") * NC + lax.axis_index("c")
    base = wid * dpw
    bufs = (rows0, rows1)
    sems = (sem0, sem1)

    def group_body(g, _):
        g0 = base + g * GROUP
        tok0 = pl.multiple_of(g0 * S, 8)
        pltpu.sync_copy(ids_hbm.at[pl.ds(tok0, GROUP * S)], ids_v)
        pltpu.sync_copy(w_hbm.at[pl.ds(tok0, GROUP * S)], w_v)

        def doc_body(d, _):
            for i in range(D // L):
                acc[pl.ds(i * L, L)] = jnp.zeros((L,), jnp.float32)

            def start(ci):
                off, sz = CHUNKS[ci]
                return pltpu.async_copy(
                    wt_hbm.at[ids_v.at[pl.ds(pl.multiple_of(d * S + off, 8), sz)]],
                    bufs[ci % 2].at[pl.ds(0, sz)],
                    sems[ci % 2],
                )

            descs = [None] * len(CHUNKS)
            descs[0] = start(0)
            for ci, (off, sz) in enumerate(CHUNKS):
                if ci + 1 < len(CHUNKS):
                    descs[ci + 1] = start(ci + 1)
                descs[ci].wait()
                buf = bufs[ci % 2]

                ngrp = D // (2 * L)
                halfg = ngrp // 2
                for h in range(2):
                    def row_body(r2, accs):
                        idx = jnp.full(
                            (L,), d * S + off, jnp.int32) + 2 * r2
                        wb0 = plsc.load_gather(w_v, [idx])
                        wb1 = plsc.load_gather(w_v, [idx + 1])
                        wbb0 = plsc.pack(
                            wb0, wb0, format=plsc.PackFormat.INTERLEAVED)
                        wbb1 = plsc.pack(
                            wb1, wb1, format=plsc.PackFormat.INTERLEAVED)
                        out = []
                        for i in range(halfg):
                            g = h * halfg + i
                            x0 = plsc.bitcast(
                                buf[2 * r2, pl.ds(g * L, L)], jnp.bfloat16)
                            x1 = plsc.bitcast(
                                buf[2 * r2 + 1, pl.ds(g * L, L)],
                                jnp.bfloat16)
                            p = x0 * wbb0 + x1 * wbb1
                            lo, hi = plsc.unpack(
                                p, format=plsc.PackFormat.INTERLEAVED)
                            out.append(accs[2 * i] + lo)
                            out.append(accs[2 * i + 1] + hi)
                        return tuple(out)

                    zero = jnp.zeros((L,), jnp.float32)
                    accs = lax.fori_loop(0, sz // 2, row_body, (zero,) * ngrp)
                    for i in range(halfg):
                        g = h * halfg + i
                        plsc.addupdate(
                            acc.at[pl.ds(g * L, L)], accs[2 * i])
                        plsc.addupdate(
                            acc.at[pl.ds(D // 2 + g * L, L)], accs[2 * i + 1])

            pltpu.sync_copy(
                acc, out_hbm.at[pl.ds(pl.multiple_of((g0 + d) * D, 8), D)])
            return 0

        lax.fori_loop(0, GROUP, doc_body, 0)
        return 0

    lax.fori_loop(0, dpw // GROUP, group_body, 0)

  return _sc_body


def _sc_accumulate(wt, input_ids, w):
    nb = input_ids.shape[0]
    mesh = plsc.VectorSubcoreMesh(
        core_axis_name="c", subcore_axis_name="s",
        num_cores=NC, num_subcores=NS,
    )
    f = functools.partial(
        pl.kernel,
        out_type=jax.ShapeDtypeStruct((nb * D,), jnp.float32),
        mesh=mesh,
        compiler_params=pltpu.CompilerParams(needs_layout_passes=False),
        scratch_types=[
            pltpu.VMEM((GROUP * S,), jnp.int32),
            pltpu.VMEM((GROUP * S,), jnp.float32),
            pltpu.VMEM((CMAX, D // 2), jnp.int32),
            pltpu.VMEM((CMAX, D // 2), jnp.int32),
            pltpu.VMEM((D,), jnp.float32),
            pltpu.SemaphoreType.DMA,
            pltpu.SemaphoreType.DMA,
        ],
    )(_make_sc_body(nb // NW))
    return f(wt, input_ids.reshape(-1), w.reshape(-1)).reshape(nb, D)


def _norm_body(a_ref, u_ref, o_ref):
    t = a_ref[...] + EPS * u_ref[...]
    ss = jnp.sum(t * t, axis=1, keepdims=True)
    n = jnp.maximum(jnp.sqrt(ss), 1e-12)
    o_ref[...] = t / n


def _normalize(acc, u):
    return pl.pallas_call(
        _norm_body,
        grid=(acc.shape[0] // RBLK,),
        in_specs=[
            pl.BlockSpec((RBLK, D), lambda i: (i, 0)),
            pl.BlockSpec((1, D), lambda i: (0, 0)),
        ],
        out_specs=pl.BlockSpec((RBLK, D), lambda i: (i, 0)),
        out_shape=jax.ShapeDtypeStruct(acc.shape, jnp.float32),
    )(acc, u)


def kernel(input_ids, attention_mask, W):
    ids = input_ids.astype(jnp.int32)
    msk = attention_mask.astype(jnp.int32)
    wt, ulo, uhi = _packtable(W)
    u = jnp.concatenate([ulo, uhi], axis=0).reshape(1, D)
    H = B // 2
    w0 = _weights(ids[:H], msk[:H])
    w1 = _weights(ids[H:], msk[H:])
    acc0 = _sc_accumulate(wt, ids[:H], w0)
    acc1 = _sc_accumulate(wt, ids[H:], w1)
    out0 = _normalize(acc0, u)
    out1 = _normalize(acc1, u)
    return jnp.concatenate([out0, out1], axis=0)


# speedup vs baseline: 1.2193x; 1.0726x over previous
import functools

import jax
import jax.numpy as jnp
from jax import lax
from jax.experimental import pallas as pl
from jax.experimental.pallas import tpu as pltpu
from jax.experimental.pallas import tpu_sc as plsc

B, S = 4096, 200
VOCAB = 30000
D = 768
K1 = 1.2
BB = 0.75
EPS = 1e-10

NC, NS, L = 2, 16, 16
NW = NC * NS
DPW = B // NW
GROUP = 16
CHUNKS = ((0, 56), (56, 56), (112, 56), (168, 32))
CMAX = 56


RBLK = 256
WBLK = 32


def _weights_body(ids_ref, mask_ref, w_ref):
    ids = ids_ref[...]
    msk = mask_ref[...]
    valid = (msk == 1) & (ids > 100) & (ids < VOCAB)
    vf = valid.astype(jnp.float32)
    doc_len = jnp.sum(vf, axis=1, keepdims=True)
    denom = jnp.maximum(1.0 + BB * (doc_len / 100.0 - 1.0), 0.5)
    safe = jnp.where(valid, ids, -1)
    eq = safe[:, :, None] == safe[:, None, :]
    c = jnp.sum(eq.astype(jnp.float32), axis=2)
    w_ref[...] = jnp.where(valid, (K1 + 1.0) / (c + K1 * denom), 0.0)


def _weights(input_ids, attention_mask):
    return pl.pallas_call(
        _weights_body,
        grid=(input_ids.shape[0] // WBLK,),
        in_specs=[
            pl.BlockSpec((WBLK, S), lambda i: (i, 0)),
            pl.BlockSpec((WBLK, S), lambda i: (i, 0)),
        ],
        out_specs=pl.BlockSpec((WBLK, S), lambda i: (i, 0)),
        out_shape=jax.ShapeDtypeStruct(input_ids.shape, jnp.float32),
    )(input_ids, attention_mask)


PBLK = 512


def _pack_body(wlo_ref, whi_ref, wt_ref, ulo_ref, uhi_ref):
    @pl.when(pl.program_id(0) == 0)
    def _():
        ulo_ref[...] = jnp.zeros_like(ulo_ref)
        uhi_ref[...] = jnp.zeros_like(uhi_ref)

    wlo = wlo_ref[...]
    whi = whi_ref[...]
    col = pl.program_id(0) * PBLK + jax.lax.broadcasted_iota(
        jnp.int32, (D // 2, PBLK), 1)
    mask = col < VOCAB
    ulo_ref[...] += jnp.sum(jnp.where(mask, wlo, 0.0), axis=1, keepdims=True)
    uhi_ref[...] += jnp.sum(jnp.where(mask, whi, 0.0), axis=1, keepdims=True)
    lo = lax.bitcast_convert_type(
        wlo.astype(jnp.bfloat16), jnp.uint16).astype(jnp.int32)
    hi = lax.bitcast_convert_type(
        whi.astype(jnp.bfloat16), jnp.uint16).astype(jnp.int32)
    word = lo | (hi << 16)
    wt_ref[...] = word.T


def _packtable(W):
    return pl.pallas_call(
        _pack_body,
        grid=(pl.cdiv(VOCAB, PBLK),),
        in_specs=[
            pl.BlockSpec((D // 2, PBLK), lambda i: (0, i)),
            pl.BlockSpec((D // 2, PBLK), lambda i: (1, i)),
        ],
        out_specs=[
            pl.BlockSpec((PBLK, D // 2), lambda i: (i, 0)),
            pl.BlockSpec((D // 2, 1), lambda i: (0, 0)),
            pl.BlockSpec((D // 2, 1), lambda i: (0, 0)),
        ],
        out_shape=[
            jax.ShapeDtypeStruct((VOCAB, D // 2), jnp.int32),
            jax.ShapeDtypeStruct((D // 2, 1), jnp.float32),
            jax.ShapeDtypeStruct((D // 2, 1), jnp.float32),
        ],
    )(W, W)


def _make_sc_body(dpw):
  def _sc_body(wt_hbm, ids_hbm, w_hbm, out_hbm,
               ids_v, w_v, rows0, rows1, acc, sem0, sem1):
    wid = lax.axis_index("s") * NC + lax.axis_index("c")
    base = wid * dpw
    bufs = (rows0, rows1)
    sems = (sem0, sem1)

    def group_body(g, _):
        g0 = base + g * GROUP
        tok0 = pl.multiple_of(g0 * S, 8)
        pltpu.sync_copy(ids_hbm.at[pl.ds(tok0, GROUP * S)], ids_v)
        pltpu.sync_copy(w_hbm.at[pl.ds(tok0, GROUP * S)], w_v)

        def doc_body(d, _):
            for i in range(D // L):
                acc[pl.ds(i * L, L)] = jnp.zeros((L,), jnp.float32)

            def start(ci):
                off, sz = CHUNKS[ci]
                return pltpu.async_copy(
                    wt_hbm.at[ids_v.at[pl.ds(pl.multiple_of(d * S + off, 8), sz)]],
                    bufs[ci % 2].at[pl.ds(0, sz)],
                    sems[ci % 2],
                )

            descs = [None] * len(CHUNKS)
            descs[0] = start(0)
            for ci, (off, sz) in enumerate(CHUNKS):
                if ci + 1 < len(CHUNKS):
                    descs[ci + 1] = start(ci + 1)
                descs[ci].wait()
                buf = bufs[ci % 2]

                ngrp = D // (2 * L)
                halfg = ngrp // 2
                for h in range(2):
                    def row_body(r2, accs):
                        idx = jnp.full(
                            (L,), d * S + off, jnp.int32) + 2 * r2
                        wb0 = plsc.load_gather(w_v, [idx])
                        wb1 = plsc.load_gather(w_v, [idx + 1])
                        wbb0 = plsc.pack(
                            wb0, wb0, format=plsc.PackFormat.INTERLEAVED)
                        wbb1 = plsc.pack(
                            wb1, wb1, format=plsc.PackFormat.INTERLEAVED)
                        out = []
                        for i in range(halfg):
                            g = h * halfg + i
                            x0 = plsc.bitcast(
                                buf[2 * r2, pl.ds(g * L, L)], jnp.bfloat16)
                            x1 = plsc.bitcast(
                                buf[2 * r2 + 1, pl.ds(g * L, L)],
                                jnp.bfloat16)
                            p = x0 * wbb0 + x1 * wbb1
                            lo, hi = plsc.unpack(
                                p, format=plsc.PackFormat.INTERLEAVED)
                            out.append(accs[2 * i] + lo)
                            out.append(accs[2 * i + 1] + hi)
                        return tuple(out)

                    zero = jnp.zeros((L,), jnp.float32)
                    accs = lax.fori_loop(0, sz // 2, row_body, (zero,) * ngrp)
                    for i in range(halfg):
                        g = h * halfg + i
                        plsc.addupdate(
                            acc.at[pl.ds(g * L, L)], accs[2 * i])
                        plsc.addupdate(
                            acc.at[pl.ds(D // 2 + g * L, L)], accs[2 * i + 1])

            pltpu.sync_copy(
                acc, out_hbm.at[pl.ds(pl.multiple_of((g0 + d) * D, 8), D)])
            return 0

        lax.fori_loop(0, GROUP, doc_body, 0)
        return 0

    lax.fori_loop(0, dpw // GROUP, group_body, 0)

  return _sc_body


def _sc_accumulate(wt, input_ids, w):
    nb = input_ids.shape[0]
    mesh = plsc.VectorSubcoreMesh(
        core_axis_name="c", subcore_axis_name="s",
        num_cores=NC, num_subcores=NS,
    )
    f = functools.partial(
        pl.kernel,
        out_type=jax.ShapeDtypeStruct((nb * D,), jnp.float32),
        mesh=mesh,
        compiler_params=pltpu.CompilerParams(needs_layout_passes=False),
        scratch_types=[
            pltpu.VMEM((GROUP * S,), jnp.int32),
            pltpu.VMEM((GROUP * S,), jnp.float32),
            pltpu.VMEM((CMAX, D // 2), jnp.int32),
            pltpu.VMEM((CMAX, D // 2), jnp.int32),
            pltpu.VMEM((D,), jnp.float32),
            pltpu.SemaphoreType.DMA,
            pltpu.SemaphoreType.DMA,
        ],
    )(_make_sc_body(nb // NW))
    return f(wt, input_ids.reshape(-1), w.reshape(-1)).reshape(nb, D)


def _norm_body(a_ref, u_ref, o_ref):
    t = a_ref[...] + EPS * u_ref[...]
    ss = jnp.sum(t * t, axis=1, keepdims=True)
    n = jnp.maximum(jnp.sqrt(ss), 1e-12)
    o_ref[...] = t / n


def _normalize(acc, u):
    return pl.pallas_call(
        _norm_body,
        grid=(acc.shape[0] // RBLK,),
        in_specs=[
            pl.BlockSpec((RBLK, D), lambda i: (i, 0)),
            pl.BlockSpec((1, D), lambda i: (0, 0)),
        ],
        out_specs=pl.BlockSpec((RBLK, D), lambda i: (i, 0)),
        out_shape=jax.ShapeDtypeStruct(acc.shape, jnp.float32),
    )(acc, u)


def kernel(input_ids, attention_mask, W):
    ids = input_ids.astype(jnp.int32)
    msk = attention_mask.astype(jnp.int32)
    wt, ulo, uhi = _packtable(W)
    u = jnp.concatenate([ulo, uhi], axis=0).reshape(1, D)
    NSLC = 4
    H = B // NSLC
    ws = [_weights(ids[i * H:(i + 1) * H], msk[i * H:(i + 1) * H])
          for i in range(NSLC)]
    accs = [_sc_accumulate(wt, ids[i * H:(i + 1) * H], ws[i])
            for i in range(NSLC)]
    outs = [_normalize(a, u) for a in accs]
    return jnp.concatenate(outs, axis=0)


# speedup vs baseline: 1.2389x; 1.0160x over previous
import functools

import jax
import jax.numpy as jnp
from jax import lax
from jax.experimental import pallas as pl
from jax.experimental.pallas import tpu as pltpu
from jax.experimental.pallas import tpu_sc as plsc

B, S = 4096, 200
VOCAB = 30000
D = 768
K1 = 1.2
BB = 0.75
EPS = 1e-10

NC, NS, L = 2, 16, 16
NW = NC * NS
DPW = B // NW
GROUP = 16
CHUNKS = ((0, 56), (56, 56), (112, 56), (168, 32))
CMAX = 56


RBLK = 256
WBLK = 32


def _weights_body(ids_ref, mask_ref, w_ref):
    ids = ids_ref[...]
    msk = mask_ref[...]
    valid = (msk == 1) & (ids > 100) & (ids < VOCAB)
    vf = valid.astype(jnp.float32)
    doc_len = jnp.sum(vf, axis=1, keepdims=True)
    denom = jnp.maximum(1.0 + BB * (doc_len / 100.0 - 1.0), 0.5)
    safe = jnp.where(valid, ids, -1)
    eq = safe[:, :, None] == safe[:, None, :]
    c = jnp.sum(eq.astype(jnp.float32), axis=2)
    w_ref[...] = jnp.where(valid, (K1 + 1.0) / (c + K1 * denom), 0.0)


def _weights(input_ids, attention_mask):
    return pl.pallas_call(
        _weights_body,
        grid=(input_ids.shape[0] // WBLK,),
        in_specs=[
            pl.BlockSpec((WBLK, S), lambda i: (i, 0)),
            pl.BlockSpec((WBLK, S), lambda i: (i, 0)),
        ],
        out_specs=pl.BlockSpec((WBLK, S), lambda i: (i, 0)),
        out_shape=jax.ShapeDtypeStruct(input_ids.shape, jnp.float32),
    )(input_ids, attention_mask)


PBLK = 512


def _pack_body(wlo_ref, whi_ref, wt_ref, ulo_ref, uhi_ref):
    @pl.when(pl.program_id(0) == 0)
    def _():
        ulo_ref[...] = jnp.zeros_like(ulo_ref)
        uhi_ref[...] = jnp.zeros_like(uhi_ref)

    wlo = wlo_ref[...]
    whi = whi_ref[...]
    col = pl.program_id(0) * PBLK + jax.lax.broadcasted_iota(
        jnp.int32, (D // 2, PBLK), 1)
    mask = col < VOCAB
    ulo_ref[...] += jnp.sum(jnp.where(mask, wlo, 0.0), axis=1, keepdims=True)
    uhi_ref[...] += jnp.sum(jnp.where(mask, whi, 0.0), axis=1, keepdims=True)
    lo = lax.bitcast_convert_type(
        wlo.astype(jnp.bfloat16), jnp.uint16).astype(jnp.int32)
    hi = lax.bitcast_convert_type(
        whi.astype(jnp.bfloat16), jnp.uint16).astype(jnp.int32)
    word = lo | (hi << 16)
    wt_ref[...] = word.T


def _packtable(W):
    return pl.pallas_call(
        _pack_body,
        grid=(pl.cdiv(VOCAB, PBLK),),
        in_specs=[
            pl.BlockSpec((D // 2, PBLK), lambda i: (0, i)),
            pl.BlockSpec((D // 2, PBLK), lambda i: (1, i)),
        ],
        out_specs=[
            pl.BlockSpec((PBLK, D // 2), lambda i: (i, 0)),
            pl.BlockSpec((D // 2, 1), lambda i: (0, 0)),
            pl.BlockSpec((D // 2, 1), lambda i: (0, 0)),
        ],
        out_shape=[
            jax.ShapeDtypeStruct((VOCAB, D // 2), jnp.int32),
            jax.ShapeDtypeStruct((D // 2, 1), jnp.float32),
            jax.ShapeDtypeStruct((D // 2, 1), jnp.float32),
        ],
    )(W, W)


def _make_sc_body(dpw):
  def _sc_body(wt_hbm, ids_hbm, w_hbm, out_hbm,
               ids_v, w_v, rows0, rows1, acc, sem0, sem1):
    wid = lax.axis_index("s") * NC + lax.axis_index("c")
    base = wid * dpw
    bufs = (rows0, rows1)
    sems = (sem0, sem1)

    def group_body(g, _):
        g0 = base + g * GROUP
        tok0 = pl.multiple_of(g0 * S, 8)
        pltpu.sync_copy(ids_hbm.at[pl.ds(tok0, GROUP * S)], ids_v)
        pltpu.sync_copy(w_hbm.at[pl.ds(tok0, GROUP * S)], w_v)

        def doc_body(d, _):
            for i in range(D // L):
                acc[pl.ds(i * L, L)] = jnp.zeros((L,), jnp.float32)

            def start(ci):
                off, sz = CHUNKS[ci]
                return pltpu.async_copy(
                    wt_hbm.at[ids_v.at[pl.ds(pl.multiple_of(d * S + off, 8), sz)]],
                    bufs[ci % 2].at[pl.ds(0, sz)],
                    sems[ci % 2],
                )

            descs = [None] * len(CHUNKS)
            descs[0] = start(0)
            for ci, (off, sz) in enumerate(CHUNKS):
                if ci + 1 < len(CHUNKS):
                    descs[ci + 1] = start(ci + 1)
                descs[ci].wait()
                buf = bufs[ci % 2]

                ngrp = D // (2 * L)
                halfg = ngrp // 2
                for h in range(2):
                    def row_body(r2, accs):
                        idx = jnp.full(
                            (L,), d * S + off, jnp.int32) + 2 * r2
                        wb0 = plsc.load_gather(w_v, [idx])
                        wb1 = plsc.load_gather(w_v, [idx + 1])
                        wbb0 = plsc.pack(
                            wb0, wb0, format=plsc.PackFormat.INTERLEAVED)
                        wbb1 = plsc.pack(
                            wb1, wb1, format=plsc.PackFormat.INTERLEAVED)
                        out = []
                        for i in range(halfg):
                            g = h * halfg + i
                            x0 = plsc.bitcast(
                                buf[2 * r2, pl.ds(g * L, L)], jnp.bfloat16)
                            x1 = plsc.bitcast(
                                buf[2 * r2 + 1, pl.ds(g * L, L)],
                                jnp.bfloat16)
                            p = x0 * wbb0 + x1 * wbb1
                            lo, hi = plsc.unpack(
                                p, format=plsc.PackFormat.INTERLEAVED)
                            out.append(accs[2 * i] + lo)
                            out.append(accs[2 * i + 1] + hi)
                        return tuple(out)

                    zero = jnp.zeros((L,), jnp.float32)
                    accs = lax.fori_loop(0, sz // 2, row_body, (zero,) * ngrp)
                    for i in range(halfg):
                        g = h * halfg + i
                        plsc.addupdate(
                            acc.at[pl.ds(g * L, L)], accs[2 * i])
                        plsc.addupdate(
                            acc.at[pl.ds(D // 2 + g * L, L)], accs[2 * i + 1])

            pltpu.sync_copy(
                acc, out_hbm.at[pl.ds(pl.multiple_of((g0 + d) * D, 8), D)])
            return 0

        lax.fori_loop(0, GROUP, doc_body, 0)
        return 0

    lax.fori_loop(0, dpw // GROUP, group_body, 0)

  return _sc_body


def _sc_accumulate(wt, input_ids, w):
    nb = input_ids.shape[0]
    mesh = plsc.VectorSubcoreMesh(
        core_axis_name="c", subcore_axis_name="s",
        num_cores=NC, num_subcores=NS,
    )
    f = functools.partial(
        pl.kernel,
        out_type=jax.ShapeDtypeStruct((nb * D,), jnp.float32),
        mesh=mesh,
        compiler_params=pltpu.CompilerParams(needs_layout_passes=False),
        scratch_types=[
            pltpu.VMEM((GROUP * S,), jnp.int32),
            pltpu.VMEM((GROUP * S,), jnp.float32),
            pltpu.VMEM((CMAX, D // 2), jnp.int32),
            pltpu.VMEM((CMAX, D // 2), jnp.int32),
            pltpu.VMEM((D,), jnp.float32),
            pltpu.SemaphoreType.DMA,
            pltpu.SemaphoreType.DMA,
        ],
    )(_make_sc_body(nb // NW))
    return f(wt, input_ids.reshape(-1), w.reshape(-1)).reshape(nb, D)


def _norm_body(a_ref, u_ref, o_ref):
    t = a_ref[...] + EPS * u_ref[...]
    ss = jnp.sum(t * t, axis=1, keepdims=True)
    n = jnp.maximum(jnp.sqrt(ss), 1e-12)
    o_ref[...] = t / n


def _normalize(acc, u):
    return pl.pallas_call(
        _norm_body,
        grid=(acc.shape[0] // RBLK,),
        in_specs=[
            pl.BlockSpec((RBLK, D), lambda i: (i, 0)),
            pl.BlockSpec((1, D), lambda i: (0, 0)),
        ],
        out_specs=pl.BlockSpec((RBLK, D), lambda i: (i, 0)),
        out_shape=jax.ShapeDtypeStruct(acc.shape, jnp.float32),
    )(acc, u)


def kernel(input_ids, attention_mask, W):
    ids = input_ids.astype(jnp.int32)
    msk = attention_mask.astype(jnp.int32)
    wt, ulo, uhi = _packtable(W)
    u = jnp.concatenate([ulo, uhi], axis=0).reshape(1, D)
    NSLC = 8
    H = B // NSLC
    ws = [_weights(ids[i * H:(i + 1) * H], msk[i * H:(i + 1) * H])
          for i in range(NSLC)]
    accs = [_sc_accumulate(wt, ids[i * H:(i + 1) * H], ws[i])
            for i in range(NSLC)]
    outs = [_normalize(a, u) for a in accs]
    return jnp.concatenate(outs, axis=0)
